# Initial kernel scaffold; baseline (speedup 1.0000x reference)
import jax
import jax.numpy as jnp
from jax.experimental import pallas as pl


def kernel(x, conv_w, centroids):
    raise NotImplementedError("write your pallas kernel here")


# speedup vs baseline: 1.3023x; 1.3023x over previous
import jax
import jax.numpy as jnp
from jax.experimental import pallas as pl
from jax.experimental.pallas import tpu as pltpu

_EPS = 1e-12


def _vlad_body(x_ref, w_ref, c_ref, o_ref):
    xb = x_ref[0]
    nrm2 = jnp.sum(xb * xb, axis=0, keepdims=True)
    xn = xb / jnp.maximum(jnp.sqrt(nrm2), _EPS)

    logits = jnp.dot(w_ref[...], xn, preferred_element_type=jnp.float32)

    m = jnp.max(logits, axis=0, keepdims=True)
    e = jnp.exp(logits - m)
    a = e / jnp.sum(e, axis=0, keepdims=True)

    asum = jnp.sum(a, axis=1, keepdims=True)
    vlad = jax.lax.dot_general(
        a, xn, (((1,), (1,)), ((), ())),
        preferred_element_type=jnp.float32)
    vlad = vlad - asum * c_ref[...]

    rn2 = jnp.sum(vlad * vlad, axis=1, keepdims=True)
    vlad = vlad / jnp.maximum(jnp.sqrt(rn2), _EPS)

    gn2 = jnp.sum(vlad * vlad, keepdims=True)
    o_ref[0] = vlad / jnp.maximum(jnp.sqrt(gn2), _EPS)


def kernel(x, conv_w, centroids):
    N, C, H, W = x.shape
    K = conv_w.shape[0]
    S = H * W
    xf = x.reshape(N, C, S)

    out = pl.pallas_call(
        _vlad_body,
        grid=(N,),
        in_specs=[
            pl.BlockSpec((1, C, S), lambda n: (n, 0, 0)),
            pl.BlockSpec((K, C), lambda n: (0, 0)),
            pl.BlockSpec((K, C), lambda n: (0, 0)),
        ],
        out_specs=pl.BlockSpec((1, K, C), lambda n: (n, 0, 0)),
        out_shape=jax.ShapeDtypeStruct((N, K, C), jnp.float32),
        compiler_params=pltpu.CompilerParams(
            dimension_semantics=("parallel",),
        ),
    )(xf, conv_w, centroids)
    return out.reshape(N, K * C)
